# baseline (speedup 1.0000x reference)
import functools

import jax
import jax.numpy as jnp
from jax import lax
from jax.experimental import pallas as pl
from jax.experimental.pallas import tpu as pltpu
from jax.experimental.pallas import tpu_sc as plsc

N_BITS = 16
BATCH = 1024
MAX_TABLE = 1 << (N_BITS - 1)

NC = 1
NS = 16
NW = NC * NS
ROWS = BATCH // NW
GROUPS = ROWS // 16
IDX_MINOR = 128
GCHUNKS = ROWS * N_BITS // IDX_MINOR


def _mapper_body(bits_hbm, tabs_hbm, out_hbm, bits_v, idx_v, got_v, out_v, sem):
    wid = lax.axis_index("s") * NC + lax.axis_index("c")
    base = wid * ROWS

    pltpu.sync_copy(bits_hbm.at[pl.ds(base, ROWS), :], bits_v)

    lane = lax.iota(jnp.int32, 16)
    zero = lane * 0

    for g in range(GROUPS):
        rowsel = lane + g * 16
        dbase = rowsel << 4
        addr = zero
        for p in range(N_BITS):
            flat = addr + p * MAX_TABLE
            plsc.store_scatter(idx_v, [dbase + (15 - p)], flat)
            if p < N_BITS - 1:
                col = plsc.load_gather(bits_v, [rowsel, zero + (15 - p)])
                addr = addr + (col << p)

    pltpu.async_copy(tabs_hbm.at[idx_v], got_v, sem).wait()

    for r in range(ROWS):
        bf = bits_v[r, :].astype(jnp.float32)
        t = got_v[pl.ds(r * 16, 16)]
        out_v[r, :] = bf + t - 2.0 * bf * t

    pltpu.sync_copy(out_v, out_hbm.at[pl.ds(base, ROWS), :])


@functools.cache
def _build_mapper():
    return functools.partial(
        pl.kernel,
        out_type=jax.ShapeDtypeStruct((BATCH, N_BITS), jnp.float32),
        mesh=plsc.VectorSubcoreMesh(
            core_axis_name="c", subcore_axis_name="s", num_cores=NC
        ),
        compiler_params=pltpu.CompilerParams(
            needs_layout_passes=False,
            disable_bounds_checks=True,
            disable_semaphore_checks=True,
            skip_device_barrier=True,
        ),
        scratch_types=[
            pltpu.VMEM((ROWS, N_BITS), jnp.int32),
            pltpu.VMEM((ROWS * N_BITS,), jnp.int32),
            pltpu.VMEM((ROWS * N_BITS,), jnp.float32),
            pltpu.VMEM((ROWS, N_BITS), jnp.float32),
            pltpu.SemaphoreType.DMA,
        ],
    )(_mapper_body)


def kernel(bits, tables):
    return _build_mapper()(bits, tables.reshape(-1))


# speedup vs baseline: 1.0087x; 1.0087x over previous
import functools

import jax
import jax.numpy as jnp
from jax import lax
from jax.experimental import pallas as pl
from jax.experimental.pallas import tpu as pltpu
from jax.experimental.pallas import tpu_sc as plsc

N_BITS = 16
BATCH = 1024
MAX_TABLE = 1 << (N_BITS - 1)

NC = 1
NS = 16
NW = NC * NS
ROWS = BATCH // NW
GROUPS = ROWS // 16
IDX_MINOR = 128
GCHUNKS = ROWS * N_BITS // IDX_MINOR


def _mapper_body(bits_hbm, tabs_hbm, out_hbm, bits_v, idx_v, got_v, out_v, sem):
    wid = lax.axis_index("s") * NC + lax.axis_index("c")
    base = wid * ROWS

    pltpu.sync_copy(bits_hbm.at[pl.ds(base, ROWS), :], bits_v)

    lane = lax.iota(jnp.int32, 16)
    zero = lane * 0

    def g_body(g, carry):
        rowsel = lane + g * 16
        dbase = rowsel << 4

        def p_body(p, addr):
            flat = addr + p * MAX_TABLE
            plsc.store_scatter(idx_v, [dbase + (15 - p)], flat)
            col = plsc.load_gather(bits_v, [rowsel, zero + (15 - p)])
            return addr + (col << p)

        lax.fori_loop(0, N_BITS, p_body, zero)
        return carry

    lax.fori_loop(0, GROUPS, g_body, 0)

    pltpu.async_copy(tabs_hbm.at[idx_v], got_v, sem).wait()

    def r_body(r, carry):
        bf = bits_v[r, :].astype(jnp.float32)
        t = got_v[pl.ds(r * 16, 16)]
        out_v[r, :] = bf + t - 2.0 * bf * t
        return carry

    lax.fori_loop(0, ROWS, r_body, 0)

    pltpu.sync_copy(out_v, out_hbm.at[pl.ds(base, ROWS), :])


@functools.cache
def _build_mapper():
    return functools.partial(
        pl.kernel,
        out_type=jax.ShapeDtypeStruct((BATCH, N_BITS), jnp.float32),
        mesh=plsc.VectorSubcoreMesh(
            core_axis_name="c", subcore_axis_name="s", num_cores=NC
        ),
        compiler_params=pltpu.CompilerParams(
            needs_layout_passes=False,
            disable_bounds_checks=True,
            disable_semaphore_checks=True,
            skip_device_barrier=True,
        ),
        scratch_types=[
            pltpu.VMEM((ROWS, N_BITS), jnp.int32),
            pltpu.VMEM((ROWS * N_BITS,), jnp.int32),
            pltpu.VMEM((ROWS * N_BITS,), jnp.float32),
            pltpu.VMEM((ROWS, N_BITS), jnp.float32),
            pltpu.SemaphoreType.DMA,
        ],
    )(_mapper_body)


def kernel(bits, tables):
    return _build_mapper()(bits, tables.reshape(-1))


# speedup vs baseline: 1.0111x; 1.0024x over previous
import functools

import jax
import jax.numpy as jnp
from jax import lax
from jax.experimental import pallas as pl
from jax.experimental.pallas import tpu as pltpu
from jax.experimental.pallas import tpu_sc as plsc

N_BITS = 16
BATCH = 1024
MAX_TABLE = 1 << (N_BITS - 1)

NC = 1
NS = 16
NW = 8
ROWS = BATCH // NW
GROUPS = ROWS // 16


def _mapper_body(bits_hbm, tabs_hbm, out_hbm, bits_v, idx_v, got_v, out_v, sem):
    wid = lax.axis_index("s") * NC + lax.axis_index("c")
    base = wid * ROWS

    @pl.when(wid < NW)
    def _active_worker():
        _worker(bits_hbm, tabs_hbm, out_hbm, bits_v, idx_v, got_v, out_v, sem, base)


def _worker(bits_hbm, tabs_hbm, out_hbm, bits_v, idx_v, got_v, out_v, sem, base):
    pltpu.sync_copy(bits_hbm.at[:, pl.ds(base, ROWS)], bits_v)

    lane = lax.iota(jnp.int32, 16)
    zero = lane * 0

    def idx_body(g, carry):
        gbase = g * 16

        def p_body(p, addr):
            phys = (
                ((p >> 3) << 18)
                + ((addr >> 7) << 10)
                + ((p & 7) << 7)
                + (addr & 127)
            )
            idx_v[pl.ds(p * ROWS + gbase, 16)] = phys
            col = bits_v[15 - p, pl.ds(gbase, 16)]
            return addr + (col << p)

        lax.fori_loop(0, N_BITS, p_body, zero)
        return carry

    lax.fori_loop(0, GROUPS, idx_body, 0)

    pltpu.async_copy(tabs_hbm.at[idx_v], got_v, sem).wait()

    def out_body(g, carry):
        gbase = g * 16

        def c_body(c, carry2):
            bf = bits_v[c, pl.ds(gbase, 16)].astype(jnp.float32)
            t = got_v[pl.ds((15 - c) * ROWS + gbase, 16)]
            out_v[c, pl.ds(gbase, 16)] = bf + t - 2.0 * bf * t
            return carry2

        lax.fori_loop(0, N_BITS, c_body, carry)
        return carry

    lax.fori_loop(0, GROUPS, out_body, 0)

    pltpu.sync_copy(out_v, out_hbm.at[:, pl.ds(base, ROWS)])


@functools.cache
def _build_mapper():
    return functools.partial(
        pl.kernel,
        out_type=jax.ShapeDtypeStruct((N_BITS, BATCH), jnp.float32),
        mesh=plsc.VectorSubcoreMesh(
            core_axis_name="c", subcore_axis_name="s", num_cores=NC
        ),
        compiler_params=pltpu.CompilerParams(
            needs_layout_passes=False,
            disable_bounds_checks=True,
            disable_semaphore_checks=True,
            skip_device_barrier=True,
        ),
        scratch_types=[
            pltpu.VMEM((N_BITS, ROWS), jnp.int32),
            pltpu.VMEM((N_BITS * ROWS,), jnp.int32),
            pltpu.VMEM((N_BITS * ROWS,), jnp.float32),
            pltpu.VMEM((N_BITS, ROWS), jnp.float32),
            pltpu.SemaphoreType.DMA,
        ],
    )(_mapper_body)


def kernel(bits, tables):
    bits_t = bits.T
    tabs_flat = tables.reshape(2, 8, 256, 128).transpose(0, 2, 1, 3).reshape(-1)
    out_t = _build_mapper()(bits_t, tabs_flat)
    return out_t.T
